# Initial kernel scaffold; baseline (speedup 1.0000x reference)
import jax
import jax.numpy as jnp
from jax.experimental import pallas as pl


def kernel(tokens, input_embedding_table):
    raise NotImplementedError("write your pallas kernel here")


# speedup vs baseline: 2.6082x; 2.6082x over previous
import functools

import jax
import jax.numpy as jnp
from jax import lax
from jax.experimental import pallas as pl
from jax.experimental.pallas import tpu as pltpu
from jax.experimental.pallas import tpu_sc as plsc

VOCAB = 100000
EMBED = 64
B = 1024
L = 200
SCALE = 8.0

NC = 2
NS = 16
NW = NC * NS
TOT = B * L
BPW = TOT // NW
CH = 128
NCH = BPW // CH

_mesh = plsc.VectorSubcoreMesh(core_axis_name="c", subcore_axis_name="s")


@functools.partial(
    pl.kernel,
    mesh=_mesh,
    out_type=jax.ShapeDtypeStruct((TOT, EMBED), jnp.float32),
    scratch_types=[
        pltpu.VMEM((NCH, CH), jnp.int32),
        pltpu.VMEM((CH, EMBED), jnp.float32),
        pltpu.SemaphoreType.DMA,
    ],
    compiler_params=pltpu.CompilerParams(use_tc_tiling_on_sc=False),
)
def _embed_gather(idx_hbm, table_hbm, out_hbm, idx_v, rows_v, sem):
    wid = lax.axis_index("s") * NC + lax.axis_index("c")
    pltpu.sync_copy(idx_hbm.at[wid], idx_v)
    row_base = wid * BPW

    def chunk_body(c, _):
        pltpu.async_copy(table_hbm.at[idx_v.at[c]], rows_v, sem).wait()

        def scale_body(i, _):
            for j in range(EMBED // 16):
                sl = pl.ds(j * 16, 16)
                rows_v[i, sl] = rows_v[i, sl] * SCALE
            return 0

        lax.fori_loop(0, CH, scale_body, 0)

        pltpu.sync_copy(rows_v, out_hbm.at[pl.ds(row_base + c * CH, CH)])
        return 0

    lax.fori_loop(0, NCH, chunk_body, 0)


def kernel(tokens, input_embedding_table):
    idx = tokens.reshape(NW, NCH, CH).astype(jnp.int32)
    out = _embed_gather(idx, input_embedding_table)
    return out.reshape(B, L, EMBED)


# speedup vs baseline: 3.2504x; 1.2462x over previous
import functools

import jax
import jax.numpy as jnp
from jax import lax
from jax.experimental import pallas as pl
from jax.experimental.pallas import tpu as pltpu
from jax.experimental.pallas import tpu_sc as plsc

VOCAB = 100000
EMBED = 64
B = 1024
L = 200
SCALE = 8.0

NC = 2
NS = 16
NW = NC * NS
TOT = B * L
BPW = TOT // NW
CH = 128
NCH = BPW // CH
G = 5
GR = G * CH
NG = BPW // GR
RPI = 8

_mesh = plsc.VectorSubcoreMesh(core_axis_name="c", subcore_axis_name="s")


def _scale_buf(buf):
    def body(i, _):
        for r in range(RPI):
            row = i * RPI + r
            for j in range(EMBED // 16):
                sl = pl.ds(j * 16, 16)
                buf[row, sl] = buf[row, sl] * SCALE
        return 0

    lax.fori_loop(0, GR // RPI, body, 0)


@functools.partial(
    pl.kernel,
    mesh=_mesh,
    out_type=jax.ShapeDtypeStruct((TOT, EMBED), jnp.float32),
    scratch_types=[
        pltpu.VMEM((NCH, CH), jnp.int32),
        pltpu.VMEM((GR, EMBED), jnp.float32),
        pltpu.VMEM((GR, EMBED), jnp.float32),
        pltpu.SemaphoreType.DMA,
        pltpu.SemaphoreType.DMA,
        pltpu.SemaphoreType.DMA,
        pltpu.SemaphoreType.DMA,
    ],
    compiler_params=pltpu.CompilerParams(use_tc_tiling_on_sc=False),
)
def _embed_gather(idx_hbm, table_hbm, out_hbm, idx_v, buf0, buf1,
                  gsem0, gsem1, osem0, osem1):
    wid = lax.axis_index("s") * NC + lax.axis_index("c")
    pltpu.sync_copy(idx_hbm.at[wid], idx_v)
    row_base = wid * BPW

    bufs = (buf0, buf1)
    gsems = (gsem0, gsem1)
    osems = (osem0, osem1)
    ghandles = {}
    ohandles = {}

    for g in range(NG + 1):
        b = g % 2
        if g < NG:
            if g >= 2:
                ohandles[g - 2].wait()
            hs = []
            for k in range(G):
                hs.append(pltpu.async_copy(
                    table_hbm.at[idx_v.at[g * G + k]],
                    bufs[b].at[pl.ds(k * CH, CH)],
                    gsems[b]))
            ghandles[g] = hs
        if g >= 1:
            gp = g - 1
            bp = gp % 2
            for h in ghandles[gp]:
                h.wait()
            _scale_buf(bufs[bp])
            ohandles[gp] = pltpu.async_copy(
                bufs[bp],
                out_hbm.at[pl.ds(row_base + gp * GR, GR)],
                osems[bp])

    ohandles[NG - 2].wait()
    ohandles[NG - 1].wait()


def kernel(tokens, input_embedding_table):
    idx = tokens.reshape(NW, NCH, CH).astype(jnp.int32)
    out = _embed_gather(idx, input_embedding_table)
    return out.reshape(B, L, EMBED)


# speedup vs baseline: 3.2530x; 1.0008x over previous
import functools

import jax
import jax.numpy as jnp
from jax import lax
from jax.experimental import pallas as pl
from jax.experimental.pallas import tpu as pltpu
from jax.experimental.pallas import tpu_sc as plsc

VOCAB = 100000
EMBED = 64
B = 1024
L = 200
SCALE = 8.0

NC = 2
NS = 16
NW = NC * NS
TOT = B * L
BPW = TOT // NW
CH = 128
NCH = BPW // CH
G = 5
GR = G * CH
NG = BPW // GR
RPI = 8

_mesh = plsc.VectorSubcoreMesh(core_axis_name="c", subcore_axis_name="s")


def _scale_buf(buf):
    @plsc.parallel_loop(0, GR, step=RPI, unroll=2)
    def _(i):
        for r in range(RPI):
            for j in range(EMBED // 16):
                sl = pl.ds(j * 16, 16)
                buf[i + r, sl] = buf[i + r, sl] * SCALE


@functools.partial(
    pl.kernel,
    mesh=_mesh,
    out_type=jax.ShapeDtypeStruct((TOT, EMBED), jnp.float32),
    scratch_types=[
        pltpu.VMEM((NCH, CH), jnp.int32),
        pltpu.VMEM((GR, EMBED), jnp.float32),
        pltpu.VMEM((GR, EMBED), jnp.float32),
        pltpu.SemaphoreType.DMA,
        pltpu.SemaphoreType.DMA,
        pltpu.SemaphoreType.DMA,
        pltpu.SemaphoreType.DMA,
    ],
    compiler_params=pltpu.CompilerParams(use_tc_tiling_on_sc=False),
)
def _embed_gather(idx_hbm, table_hbm, out_hbm, idx_v, buf0, buf1,
                  gsem0, gsem1, osem0, osem1):
    wid = lax.axis_index("s") * NC + lax.axis_index("c")
    pltpu.sync_copy(idx_hbm.at[wid], idx_v)
    row_base = wid * BPW

    bufs = (buf0, buf1)
    gsems = (gsem0, gsem1)
    osems = (osem0, osem1)
    ghandles = {}
    ohandles = {}

    for g in range(NG + 1):
        b = g % 2
        if g < NG:
            if g >= 2:
                ohandles[g - 2].wait()
            hs = []
            for k in range(G):
                hs.append(pltpu.async_copy(
                    table_hbm.at[idx_v.at[g * G + k]],
                    bufs[b].at[pl.ds(k * CH, CH)],
                    gsems[b]))
            ghandles[g] = hs
        if g >= 1:
            gp = g - 1
            bp = gp % 2
            for h in ghandles[gp]:
                h.wait()
            _scale_buf(bufs[bp])
            ohandles[gp] = pltpu.async_copy(
                bufs[bp],
                out_hbm.at[pl.ds(row_base + gp * GR, GR)],
                osems[bp])

    ohandles[NG - 2].wait()
    ohandles[NG - 1].wait()


def kernel(tokens, input_embedding_table):
    idx = tokens.reshape(NW, NCH, CH).astype(jnp.int32)
    out = _embed_gather(idx, input_embedding_table)
    return out.reshape(B, L, EMBED)
